# baseline (speedup 1.0000x reference)
import functools

import jax
import jax.numpy as jnp
import numpy as np
from jax import lax
from jax.experimental import pallas as pl
from jax.experimental.pallas import tpu as pltpu
from jax.experimental.pallas import tpu_sc as plsc

B = 4
N_POINT = 4096
N_CENTER = 1024
N_NEAR = 32
C_IN = 128
C_MID = 183
C_OUT = 256

BM = 512
M_TOTAL = B * N_CENTER * N_NEAR
N_ROWS_F = float(M_TOTAL)
EPS = 1e-5


_FPS_SUB = 32
_FPS_LANE = 128


def _fps_body(x_ref, y_ref, z_ref, idx_out_ref, ctr_out_ref):
    X = x_ref[...]
    Y = y_ref[...]
    Z = z_ref[...]
    niota = (jax.lax.broadcasted_iota(jnp.int32, (B, _FPS_SUB, _FPS_LANE), 1) * _FPS_LANE
             + jax.lax.broadcasted_iota(jnp.int32, (B, _FPS_SUB, _FPS_LANE), 2))

    def step(t, carry):
        dist, far = carry
        cmask = niota == far
        cx = jnp.min(jnp.where(cmask, X, 1e9), axis=(1, 2), keepdims=True)
        cy = jnp.min(jnp.where(cmask, Y, 1e9), axis=(1, 2), keepdims=True)
        cz = jnp.min(jnp.where(cmask, Z, 1e9), axis=(1, 2), keepdims=True)
        idx_out_ref[pl.ds(t, 1), :] = far[:, 0, :].reshape(1, B)
        ctr_out_ref[pl.ds(t, 1), :] = jnp.concatenate(
            [cx[:, 0, :], cy[:, 0, :], cz[:, 0, :]], axis=0).reshape(1, 3 * B)
        dx = X - cx
        dy = Y - cy
        dz = Z - cz
        d = (dx * dx + dz * dz) + dy * dy
        dist = jnp.minimum(dist, d)
        m = jnp.max(dist, axis=(1, 2), keepdims=True)
        far = jnp.min(jnp.where(dist == m, niota, jnp.int32(N_POINT)),
                      axis=(1, 2), keepdims=True)
        return dist, far

    init = (jnp.full((B, _FPS_SUB, _FPS_LANE), 1e10, dtype=jnp.float32),
            jnp.zeros((B, 1, 1), dtype=jnp.int32))
    jax.lax.fori_loop(0, N_CENTER, step, init, unroll=False)


def _fps_pallas(xyz):
    xt = jnp.transpose(xyz, (2, 0, 1)).reshape(3, B, _FPS_SUB, _FPS_LANE)
    idx_bs, ctr = pl.pallas_call(
        _fps_body,
        grid=(1,),
        in_specs=[pl.BlockSpec((B, _FPS_SUB, _FPS_LANE), lambda i: (0, 0, 0))] * 3,
        out_specs=[
            pl.BlockSpec((N_CENTER, B), lambda i: (0, 0)),
            pl.BlockSpec((N_CENTER, 3 * B), lambda i: (0, 0)),
        ],
        out_shape=[
            jax.ShapeDtypeStruct((N_CENTER, B), jnp.int32),
            jax.ShapeDtypeStruct((N_CENTER, 3 * B), jnp.float32),
        ],
    )(xt[0], xt[1], xt[2])
    fps_idx = jnp.transpose(idx_bs)
    center_xyz = jnp.transpose(ctr.reshape(N_CENTER, 3, B), (2, 0, 1))
    return fps_idx, center_xyz


def _dist_body(ctr_ref, xt_ref, d_ref, tau_ref):
    lmask = (jax.lax.broadcasted_iota(jnp.int32, (_DCH, 16), 1) < 3
             ).astype(jnp.float32)
    cbf = (ctr_ref[...] * lmask).astype(jnp.bfloat16)
    xbf = xt_ref[0].astype(jnp.bfloat16)
    P2 = jnp.dot(cbf, xbf, preferred_element_type=jnp.float32)
    sqc = ctr_ref[:, 4:5]
    sqrow = xt_ref[0, 3:4, :]
    P = (sqc + sqrow) + P2
    d_ref[0] = P
    m = P[:, 0:128]
    for c in range(1, 32):
        m = jnp.minimum(m, P[:, c * 128:(c + 1) * 128])
    for k in range(31):
        mn = jnp.min(m, axis=1, keepdims=True)
        m = jnp.where(m == mn, 1e9, m)
    mn = jnp.min(m, axis=1, keepdims=True)
    tau_ref[0] = jnp.broadcast_to(mn, (_DCH, 16))


_DCH = 256


def _dist_pallas(ctrK, xtK):
    steps_per_b = N_CENTER // _DCH
    return pl.pallas_call(
        _dist_body,
        grid=(B * steps_per_b,),
        in_specs=[
            pl.BlockSpec((_DCH, 16), lambda i: (i, 0)),
            pl.BlockSpec((1, 16, N_POINT), lambda i: (i // 4, 0, 0)),
        ],
        out_specs=[
            pl.BlockSpec((1, _DCH, N_POINT), lambda i: (i // 4, i % 4, 0)),
            pl.BlockSpec((1, _DCH, 16), lambda i: (i // 4, i % 4, 0)),
        ],
        out_shape=[
            jax.ShapeDtypeStruct((B, N_CENTER, N_POINT), jnp.float32),
            jax.ShapeDtypeStruct((B, N_CENTER, 16), jnp.float32),
        ],
    )(ctrK, xtK)


_SROWS = (B * N_CENTER) // 32
_NV = N_POINT // 16
_MAXI = 2147483647
_SCAP = 128


def _vbcast_last(x):
    return plsc.cummax(lax.rev(x, dimensions=(0,)))


def _vbcast_min(x):
    r = -plsc.cummax(-x)
    return -plsc.cummax(lax.rev(-r, dimensions=(0,)))


def _sc_select(dist2, tau16):
    mesh = plsc.VectorSubcoreMesh(core_axis_name="c", subcore_axis_name="s")

    @functools.partial(
        pl.kernel,
        mesh=mesh,
        out_type=jax.ShapeDtypeStruct((B * N_CENTER, N_NEAR), jnp.int32),
        scratch_types=[
            pltpu.VMEM((N_POINT + 16,), jnp.float32),
            pltpu.VMEM((_SCAP + 16,), jnp.int32),
            pltpu.VMEM((16,), jnp.float32),
            pltpu.VMEM((N_NEAR,), jnp.int32),
        ],
    )
    def body(dist_hbm, tau_hbm, out_hbm, row_v, si_v, tau_v, win_v):
        wid = lax.axis_index("s") * 2 + lax.axis_index("c")
        row_v[pl.ds(N_POINT, 16)] = jnp.full((16,), jnp.inf, jnp.float32)

        def do_row(r, carry):
            row = wid * _SROWS + r
            pltpu.sync_copy(dist_hbm.at[row], row_v.at[pl.ds(0, N_POINT)])
            pltpu.sync_copy(tau_hbm.at[row], tau_v)
            tauv = tau_v[...]
            for t in range((_SCAP + 16) // 16):
                si_v[pl.ds(t * 16, 16)] = jnp.full((16,), N_POINT, jnp.int32)
            li = lax.iota(jnp.int32, 16)
            base = jnp.zeros((16,), jnp.int32)
            for v in range(_NV):
                vec = row_v[pl.ds(v * 16, 16)]
                msk = vec <= tauv
                pref = plsc.cumsum(msk.astype(jnp.int32))
                pos = base + pref - 1
                m2 = msk & (pos < _SCAP)
                plsc.store_scatter(si_v, [pos], li + v * 16, mask=m2)
                base = base + _vbcast_last(pref)
            svals, sidxs = [], []
            for t in range(_SCAP // 16):
                iv = si_v[pl.ds(t * 16, 16)]
                svals.append(plsc.load_gather(row_v, [iv]))
                sidxs.append(iv)
            w0 = jnp.zeros((16,), jnp.int32)
            w1 = jnp.zeros((16,), jnp.int32)
            for k in range(N_NEAR):
                acc = svals[0]
                for t in range(1, _SCAP // 16):
                    acc = jnp.minimum(acc, svals[t])
                mval = _vbcast_min(acc)
                acc2 = jnp.full((16,), _MAXI, jnp.int32)
                for t in range(_SCAP // 16):
                    acc2 = jnp.minimum(
                        acc2, jnp.where(svals[t] == mval, sidxs[t], jnp.int32(_MAXI)))
                nstar = _vbcast_min(acc2)
                for t in range(_SCAP // 16):
                    svals[t] = jnp.where(sidxs[t] == nstar, jnp.inf, svals[t])
                w0 = jnp.where(li == k, nstar, w0)
                w1 = jnp.where(li == k - 16, nstar, w1)
            win_v[pl.ds(0, 16)] = w0
            win_v[pl.ds(16, 16)] = w1
            pltpu.sync_copy(win_v, out_hbm.at[row])
            return carry

        lax.fori_loop(0, _SROWS, do_row, 0)

    return body(dist2, tau16)


_GW = 32
_G_PER_W = M_TOTAL // _GW
_GCHUNK = 256
_GN_CHUNKS = _G_PER_W // _GCHUNK
_GD = C_IN + 128


def _sc_gather(gidx, table):
    mesh = plsc.VectorSubcoreMesh(core_axis_name="c", subcore_axis_name="s")

    @functools.partial(
        pl.kernel,
        mesh=mesh,
        out_type=jax.ShapeDtypeStruct((M_TOTAL, _GD), jnp.float32),
        scratch_types=[
            pltpu.VMEM((_GCHUNK,), jnp.int32),
            pltpu.VMEM((_GCHUNK, _GD), jnp.float32),
            pltpu.SemaphoreType.DMA,
        ],
    )
    def body(idx_hbm, tab_hbm, out_hbm, idx_v, rows_v, s1):
        wid = lax.axis_index("s") * 2 + lax.axis_index("c")

        def chunk(c, carry):
            base = wid * _G_PER_W + c * _GCHUNK
            pltpu.sync_copy(idx_hbm.at[pl.ds(base, _GCHUNK)], idx_v)
            pltpu.async_copy(tab_hbm.at[idx_v], rows_v, s1).wait()
            pltpu.sync_copy(rows_v, out_hbm.at[pl.ds(base, _GCHUNK)])
            return carry

        lax.fori_loop(0, _GN_CHUNKS, chunk, 0)

    return body(gidx, table)


def _knn_jax(xyz, k):
    sq = jnp.sum(xyz * xyz, axis=-1)
    dist = sq[:, :, None] + sq[:, None, :] - 2.0 * jnp.einsum('bnc,bmc->bnm', xyz, xyz)
    _, idx = jax.lax.top_k(-dist, k)
    return idx


def _fps_jax(xyz, n_center):
    b, n, _ = xyz.shape

    def step(carry, _):
        dist, far = carry
        centroid = jnp.take_along_axis(xyz, far[:, None, None].astype(jnp.int32), axis=1)
        d = jnp.sum((xyz - centroid) ** 2, axis=-1)
        dist = jnp.minimum(dist, d)
        new_far = jnp.argmax(dist, axis=-1).astype(jnp.int32)
        return (dist, new_far), far

    init = (jnp.full((b, n), 1e10, dtype=jnp.float32), jnp.zeros((b,), dtype=jnp.int32))
    _, idxs = jax.lax.scan(step, init, None, length=n_center)
    return jnp.transpose(idxs)


def _index_points(points, idx):
    return jax.vmap(lambda p, i: p[i])(points, idx)


def _mlp1_body(g_ref, ctr_ref, w1a_ref, w1b_ref, b1_ref,
               h1_ref, stats_ref):
    i = pl.program_id(0)
    gfea = g_ref[:, :C_IN]
    gx = (g_ref[:, C_IN:C_IN + 16].reshape(BM // N_NEAR, N_NEAR, 16)
          - ctr_ref[...][:, None, :]).reshape(BM, 16)
    h = (jnp.dot(gfea, w1a_ref[...], preferred_element_type=jnp.float32)
         + jnp.dot(gx, w1b_ref[...], preferred_element_type=jnp.float32)
         + b1_ref[...])
    h1_ref[...] = h
    s = jnp.sum(h, axis=0, keepdims=True)
    ss = jnp.sum(h * h, axis=0, keepdims=True)
    upd = jnp.concatenate([s, ss], axis=0)

    @pl.when(i == 0)
    def _():
        stats_ref[...] = upd

    @pl.when(i > 0)
    def _():
        stats_ref[...] += upd


def _mlp_pass1(gcomb, ctr16, W1a, W1b, b1):
    grid = (M_TOTAL // BM,)
    return pl.pallas_call(
        _mlp1_body,
        grid=grid,
        in_specs=[
            pl.BlockSpec((BM, _GD), lambda i: (i, 0)),
            pl.BlockSpec((BM // N_NEAR, 16), lambda i: (i, 0)),
            pl.BlockSpec((C_IN, C_MID), lambda i: (0, 0)),
            pl.BlockSpec((16, C_MID), lambda i: (0, 0)),
            pl.BlockSpec((1, C_MID), lambda i: (0, 0)),
        ],
        out_specs=[
            pl.BlockSpec((BM, C_MID), lambda i: (i, 0)),
            pl.BlockSpec((2, C_MID), lambda i: (0, 0)),
        ],
        out_shape=[
            jax.ShapeDtypeStruct((M_TOTAL, C_MID), jnp.float32),
            jax.ShapeDtypeStruct((2, C_MID), jnp.float32),
        ],
    )(gcomb, ctr16, W1a, W1b, b1)


def _mlp2_body(h1_ref, stats1_ref, w2_ref, b2_ref, g1_ref, be1_ref,
               maxh2_ref, stats2_ref):
    i = pl.program_id(0)
    m1 = stats1_ref[0:1, :] / N_ROWS_F
    var1 = stats1_ref[1:2, :] / N_ROWS_F - m1 * m1
    inv1 = g1_ref[...] * jax.lax.rsqrt(var1 + EPS)
    a = jnp.maximum((h1_ref[...] - m1) * inv1 + be1_ref[...], 0.0)
    h2 = jnp.dot(a, w2_ref[...], preferred_element_type=jnp.float32) + b2_ref[...]
    s = jnp.sum(h2, axis=0, keepdims=True)
    ss = jnp.sum(h2 * h2, axis=0, keepdims=True)
    upd = jnp.concatenate([s, ss], axis=0)
    maxh2_ref[...] = jnp.max(h2.reshape(BM // N_NEAR, N_NEAR, C_OUT), axis=1)

    @pl.when(i == 0)
    def _():
        stats2_ref[...] = upd

    @pl.when(i > 0)
    def _():
        stats2_ref[...] += upd


def _mlp_pass2(h1, stats1, W2, b2, g1, be1):
    grid = (M_TOTAL // BM,)
    return pl.pallas_call(
        _mlp2_body,
        grid=grid,
        in_specs=[
            pl.BlockSpec((BM, C_MID), lambda i: (i, 0)),
            pl.BlockSpec((2, C_MID), lambda i: (0, 0)),
            pl.BlockSpec((C_MID, C_OUT), lambda i: (0, 0)),
            pl.BlockSpec((1, C_OUT), lambda i: (0, 0)),
            pl.BlockSpec((1, C_MID), lambda i: (0, 0)),
            pl.BlockSpec((1, C_MID), lambda i: (0, 0)),
        ],
        out_specs=[
            pl.BlockSpec((BM // N_NEAR, C_OUT), lambda i: (i, 0)),
            pl.BlockSpec((2, C_OUT), lambda i: (0, 0)),
        ],
        out_shape=[
            jax.ShapeDtypeStruct((B * N_CENTER, C_OUT), jnp.float32),
            jax.ShapeDtypeStruct((2, C_OUT), jnp.float32),
        ],
    )(h1, stats1, W2, b2, g1, be1)


def _epi_body(x_ref, stats2_ref, g2_ref, be2_ref, o_ref):
    m2 = stats2_ref[0:1, :] / N_ROWS_F
    var2 = stats2_ref[1:2, :] / N_ROWS_F - m2 * m2
    inv2 = g2_ref[...] * jax.lax.rsqrt(var2 + EPS)
    o_ref[...] = jnp.maximum((x_ref[...] - m2) * inv2 + be2_ref[...], 0.0)


def _mlp_epilogue(maxh2, stats2, g2, be2):
    grid = (8,)
    bm = (B * N_CENTER) // 8
    return pl.pallas_call(
        _epi_body,
        grid=grid,
        in_specs=[
            pl.BlockSpec((bm, C_OUT), lambda i: (i, 0)),
            pl.BlockSpec((2, C_OUT), lambda i: (0, 0)),
            pl.BlockSpec((1, C_OUT), lambda i: (0, 0)),
            pl.BlockSpec((1, C_OUT), lambda i: (0, 0)),
        ],
        out_specs=pl.BlockSpec((bm, C_OUT), lambda i: (i, 0)),
        out_shape=jax.ShapeDtypeStruct((B * N_CENTER, C_OUT), jnp.float32),
    )(maxh2, stats2, g2, be2)


def kernel(xyz, fea, W1, b1, g1, be1, W2, b2, g2, be2):
    fps_idx, center_xyz = _fps_pallas(xyz)
    sq = jnp.sum(xyz * xyz, axis=-1)
    sqc = jnp.take_along_axis(sq, fps_idx, axis=1)
    BS = B * N_CENTER
    ctrK = jnp.concatenate([
        center_xyz.reshape(BS, 3),
        jnp.ones((BS, 1), jnp.float32),
        sqc.reshape(BS, 1),
        jnp.zeros((BS, 11), jnp.float32),
    ], axis=1)
    xtK = jnp.concatenate([
        -2.0 * jnp.transpose(xyz, (0, 2, 1)),
        sq[:, None, :],
        jnp.ones((B, 1, N_POINT), jnp.float32),
        jnp.zeros((B, 11, N_POINT), jnp.float32),
    ], axis=1)
    dmat, tau16 = _dist_pallas(ctrK, xtK)
    v1, i1 = jax.lax.top_k(-dmat.reshape(B, N_CENTER, 32, 128), N_NEAR)
    gi1 = (i1 + (jnp.arange(32, dtype=jnp.int32) * 128)[None, None, :, None]
           ).reshape(B, N_CENTER, 1024)
    _, i2 = jax.lax.top_k(v1.reshape(B, N_CENTER, 1024), N_NEAR)
    group_idx = jnp.take_along_axis(gi1, i2, axis=-1)

    gidx = (group_idx.astype(jnp.int32)
            + (jnp.arange(B, dtype=jnp.int32) * N_POINT)[:, None, None]
            ).reshape(M_TOTAL)
    fea2 = fea.reshape(B * N_POINT, C_IN)
    xyzp = jnp.pad(xyz.reshape(B * N_POINT, 3), ((0, 0), (0, 125)))
    table = jnp.concatenate([fea2, xyzp], axis=1)
    gcomb = _sc_gather(gidx, table)

    ctr16 = jnp.pad(center_xyz.reshape(B * N_CENTER, 3), ((0, 0), (0, 13)))
    W1a = W1[:C_IN]
    W1b = jnp.pad(W1[C_IN:], ((0, 13), (0, 0)))
    h1, stats1 = _mlp_pass1(gcomb, ctr16, W1a, W1b, b1[None, :])
    maxh2, stats2 = _mlp_pass2(h1, stats1, W2, b2[None, :], g1[None, :], be1[None, :])
    out = _mlp_epilogue(maxh2, stats2, g2[None, :], be2[None, :])
    return (center_xyz, out.reshape(B, N_CENTER, C_OUT))


# speedup vs baseline: 2.2927x; 2.2927x over previous
import functools

import jax
import jax.numpy as jnp
import numpy as np
from jax import lax
from jax.experimental import pallas as pl
from jax.experimental.pallas import tpu as pltpu
from jax.experimental.pallas import tpu_sc as plsc

B = 4
N_POINT = 4096
N_CENTER = 1024
N_NEAR = 32
C_IN = 128
C_MID = 183
C_OUT = 256

BM = 512
M_TOTAL = B * N_CENTER * N_NEAR
N_ROWS_F = float(M_TOTAL)
EPS = 1e-5


_FPS_SUB = 32
_FPS_LANE = 128


def _fps_body(x_ref, y_ref, z_ref, idx_out_ref, ctr_out_ref):
    X = x_ref[...]
    Y = y_ref[...]
    Z = z_ref[...]
    niota = (jax.lax.broadcasted_iota(jnp.int32, (B, _FPS_SUB, _FPS_LANE), 1) * _FPS_LANE
             + jax.lax.broadcasted_iota(jnp.int32, (B, _FPS_SUB, _FPS_LANE), 2))

    def step(t, carry):
        dist, far = carry
        cmask = niota == far
        cx = jnp.min(jnp.where(cmask, X, 1e9), axis=(1, 2), keepdims=True)
        cy = jnp.min(jnp.where(cmask, Y, 1e9), axis=(1, 2), keepdims=True)
        cz = jnp.min(jnp.where(cmask, Z, 1e9), axis=(1, 2), keepdims=True)
        idx_out_ref[pl.ds(t, 1), :] = far[:, 0, :].reshape(1, B)
        ctr_out_ref[pl.ds(t, 1), :] = jnp.concatenate(
            [cx[:, 0, :], cy[:, 0, :], cz[:, 0, :]], axis=0).reshape(1, 3 * B)
        dx = X - cx
        dy = Y - cy
        dz = Z - cz
        d = (dx * dx + dz * dz) + dy * dy
        dist = jnp.minimum(dist, d)
        m = jnp.max(dist, axis=(1, 2), keepdims=True)
        far = jnp.min(jnp.where(dist == m, niota, jnp.int32(N_POINT)),
                      axis=(1, 2), keepdims=True)
        return dist, far

    init = (jnp.full((B, _FPS_SUB, _FPS_LANE), 1e10, dtype=jnp.float32),
            jnp.zeros((B, 1, 1), dtype=jnp.int32))
    jax.lax.fori_loop(0, N_CENTER, step, init, unroll=False)


def _fps_pallas(xyz):
    xt = jnp.transpose(xyz, (2, 0, 1)).reshape(3, B, _FPS_SUB, _FPS_LANE)
    idx_bs, ctr = pl.pallas_call(
        _fps_body,
        grid=(1,),
        in_specs=[pl.BlockSpec((B, _FPS_SUB, _FPS_LANE), lambda i: (0, 0, 0))] * 3,
        out_specs=[
            pl.BlockSpec((N_CENTER, B), lambda i: (0, 0)),
            pl.BlockSpec((N_CENTER, 3 * B), lambda i: (0, 0)),
        ],
        out_shape=[
            jax.ShapeDtypeStruct((N_CENTER, B), jnp.int32),
            jax.ShapeDtypeStruct((N_CENTER, 3 * B), jnp.float32),
        ],
    )(xt[0], xt[1], xt[2])
    fps_idx = jnp.transpose(idx_bs)
    center_xyz = jnp.transpose(ctr.reshape(N_CENTER, 3, B), (2, 0, 1))
    return fps_idx, center_xyz


def _dist_body(ctr_ref, xt_ref, d_ref):
    lmask = (jax.lax.broadcasted_iota(jnp.int32, (_DCH, 16), 1) < 3
             ).astype(jnp.float32)
    cbf = (ctr_ref[...] * lmask).astype(jnp.bfloat16)
    xbf = xt_ref[0].astype(jnp.bfloat16)
    P2 = jnp.dot(cbf, xbf, preferred_element_type=jnp.float32)
    sqc = ctr_ref[:, 4:5]
    sqrow = xt_ref[0, 3:4, :]
    d_ref[0] = -((sqc + sqrow) + P2)


_DCH = 256


def _dist_pallas(ctrK, xtK):
    return pl.pallas_call(
        _dist_body,
        grid=(B * (N_CENTER // _DCH),),
        in_specs=[
            pl.BlockSpec((_DCH, 16), lambda i: (i, 0)),
            pl.BlockSpec((1, 16, N_POINT), lambda i: (i // 4, 0, 0)),
        ],
        out_specs=pl.BlockSpec((1, _DCH, N_POINT), lambda i: (i // 4, i % 4, 0)),
        out_shape=jax.ShapeDtypeStruct((B, N_CENTER, N_POINT), jnp.float32),
    )(ctrK, xtK)


_SROWS = (B * N_CENTER) // 32
_NV = N_POINT // 16
_MAXI = 2147483647
_SCAP = 128


def _vbcast_last(x):
    return plsc.cummax(lax.rev(x, dimensions=(0,)))


def _vbcast_min(x):
    r = -plsc.cummax(-x)
    return -plsc.cummax(lax.rev(-r, dimensions=(0,)))


def _sc_select(dist2, tau16):
    mesh = plsc.VectorSubcoreMesh(core_axis_name="c", subcore_axis_name="s")

    @functools.partial(
        pl.kernel,
        mesh=mesh,
        out_type=jax.ShapeDtypeStruct((B * N_CENTER, N_NEAR), jnp.int32),
        scratch_types=[
            pltpu.VMEM((N_POINT + 16,), jnp.float32),
            pltpu.VMEM((_SCAP + 16,), jnp.int32),
            pltpu.VMEM((16,), jnp.float32),
            pltpu.VMEM((N_NEAR,), jnp.int32),
        ],
    )
    def body(dist_hbm, tau_hbm, out_hbm, row_v, si_v, tau_v, win_v):
        wid = lax.axis_index("s") * 2 + lax.axis_index("c")
        row_v[pl.ds(N_POINT, 16)] = jnp.full((16,), jnp.inf, jnp.float32)

        def do_row(r, carry):
            row = wid * _SROWS + r
            pltpu.sync_copy(dist_hbm.at[row], row_v.at[pl.ds(0, N_POINT)])
            pltpu.sync_copy(tau_hbm.at[row], tau_v)
            tauv = tau_v[...]
            for t in range((_SCAP + 16) // 16):
                si_v[pl.ds(t * 16, 16)] = jnp.full((16,), N_POINT, jnp.int32)
            li = lax.iota(jnp.int32, 16)
            base = jnp.zeros((16,), jnp.int32)
            for v in range(_NV):
                vec = row_v[pl.ds(v * 16, 16)]
                msk = vec <= tauv
                pref = plsc.cumsum(msk.astype(jnp.int32))
                pos = base + pref - 1
                m2 = msk & (pos < _SCAP)
                plsc.store_scatter(si_v, [pos], li + v * 16, mask=m2)
                base = base + _vbcast_last(pref)
            svals, sidxs = [], []
            for t in range(_SCAP // 16):
                iv = si_v[pl.ds(t * 16, 16)]
                svals.append(plsc.load_gather(row_v, [iv]))
                sidxs.append(iv)
            w0 = jnp.zeros((16,), jnp.int32)
            w1 = jnp.zeros((16,), jnp.int32)
            for k in range(N_NEAR):
                acc = svals[0]
                for t in range(1, _SCAP // 16):
                    acc = jnp.minimum(acc, svals[t])
                mval = _vbcast_min(acc)
                acc2 = jnp.full((16,), _MAXI, jnp.int32)
                for t in range(_SCAP // 16):
                    acc2 = jnp.minimum(
                        acc2, jnp.where(svals[t] == mval, sidxs[t], jnp.int32(_MAXI)))
                nstar = _vbcast_min(acc2)
                for t in range(_SCAP // 16):
                    svals[t] = jnp.where(sidxs[t] == nstar, jnp.inf, svals[t])
                w0 = jnp.where(li == k, nstar, w0)
                w1 = jnp.where(li == k - 16, nstar, w1)
            win_v[pl.ds(0, 16)] = w0
            win_v[pl.ds(16, 16)] = w1
            pltpu.sync_copy(win_v, out_hbm.at[row])
            return carry

        lax.fori_loop(0, _SROWS, do_row, 0)

    return body(dist2, tau16)


_GW = 32
_G_PER_W = M_TOTAL // _GW
_GCHUNK = 256
_GN_CHUNKS = _G_PER_W // _GCHUNK
_GD = C_IN + 128


def _sc_gather(gidx, table):
    mesh = plsc.VectorSubcoreMesh(core_axis_name="c", subcore_axis_name="s")

    @functools.partial(
        pl.kernel,
        mesh=mesh,
        out_type=jax.ShapeDtypeStruct((M_TOTAL, _GD), jnp.float32),
        scratch_types=[
            pltpu.VMEM((_GCHUNK,), jnp.int32),
            pltpu.VMEM((_GCHUNK, _GD), jnp.float32),
            pltpu.SemaphoreType.DMA,
        ],
    )
    def body(idx_hbm, tab_hbm, out_hbm, idx_v, rows_v, s1):
        wid = lax.axis_index("s") * 2 + lax.axis_index("c")

        def chunk(c, carry):
            base = wid * _G_PER_W + c * _GCHUNK
            pltpu.sync_copy(idx_hbm.at[pl.ds(base, _GCHUNK)], idx_v)
            pltpu.async_copy(tab_hbm.at[idx_v], rows_v, s1).wait()
            pltpu.sync_copy(rows_v, out_hbm.at[pl.ds(base, _GCHUNK)])
            return carry

        lax.fori_loop(0, _GN_CHUNKS, chunk, 0)

    return body(gidx, table)


def _knn_jax(xyz, k):
    sq = jnp.sum(xyz * xyz, axis=-1)
    dist = sq[:, :, None] + sq[:, None, :] - 2.0 * jnp.einsum('bnc,bmc->bnm', xyz, xyz)
    _, idx = jax.lax.top_k(-dist, k)
    return idx


def _fps_jax(xyz, n_center):
    b, n, _ = xyz.shape

    def step(carry, _):
        dist, far = carry
        centroid = jnp.take_along_axis(xyz, far[:, None, None].astype(jnp.int32), axis=1)
        d = jnp.sum((xyz - centroid) ** 2, axis=-1)
        dist = jnp.minimum(dist, d)
        new_far = jnp.argmax(dist, axis=-1).astype(jnp.int32)
        return (dist, new_far), far

    init = (jnp.full((b, n), 1e10, dtype=jnp.float32), jnp.zeros((b,), dtype=jnp.int32))
    _, idxs = jax.lax.scan(step, init, None, length=n_center)
    return jnp.transpose(idxs)


def _index_points(points, idx):
    return jax.vmap(lambda p, i: p[i])(points, idx)


def _mlp1_body(g_ref, ctr_ref, w1a_ref, w1b_ref, b1_ref,
               h1_ref, stats_ref):
    i = pl.program_id(0)
    gfea = g_ref[:, :C_IN]
    gx = (g_ref[:, C_IN:C_IN + 16].reshape(BM // N_NEAR, N_NEAR, 16)
          - ctr_ref[...][:, None, :]).reshape(BM, 16)
    h = (jnp.dot(gfea, w1a_ref[...], preferred_element_type=jnp.float32)
         + jnp.dot(gx, w1b_ref[...], preferred_element_type=jnp.float32)
         + b1_ref[...])
    h1_ref[...] = h
    s = jnp.sum(h, axis=0, keepdims=True)
    ss = jnp.sum(h * h, axis=0, keepdims=True)
    upd = jnp.concatenate([s, ss], axis=0)

    @pl.when(i == 0)
    def _():
        stats_ref[...] = upd

    @pl.when(i > 0)
    def _():
        stats_ref[...] += upd


def _mlp_pass1(gcomb, ctr16, W1a, W1b, b1):
    grid = (M_TOTAL // BM,)
    return pl.pallas_call(
        _mlp1_body,
        grid=grid,
        in_specs=[
            pl.BlockSpec((BM, _GD), lambda i: (i, 0)),
            pl.BlockSpec((BM // N_NEAR, 16), lambda i: (i, 0)),
            pl.BlockSpec((C_IN, C_MID), lambda i: (0, 0)),
            pl.BlockSpec((16, C_MID), lambda i: (0, 0)),
            pl.BlockSpec((1, C_MID), lambda i: (0, 0)),
        ],
        out_specs=[
            pl.BlockSpec((BM, C_MID), lambda i: (i, 0)),
            pl.BlockSpec((2, C_MID), lambda i: (0, 0)),
        ],
        out_shape=[
            jax.ShapeDtypeStruct((M_TOTAL, C_MID), jnp.float32),
            jax.ShapeDtypeStruct((2, C_MID), jnp.float32),
        ],
    )(gcomb, ctr16, W1a, W1b, b1)


def _mlp2_body(h1_ref, stats1_ref, w2_ref, b2_ref, g1_ref, be1_ref,
               maxh2_ref, stats2_ref):
    i = pl.program_id(0)
    m1 = stats1_ref[0:1, :] / N_ROWS_F
    var1 = stats1_ref[1:2, :] / N_ROWS_F - m1 * m1
    inv1 = g1_ref[...] * jax.lax.rsqrt(var1 + EPS)
    a = jnp.maximum((h1_ref[...] - m1) * inv1 + be1_ref[...], 0.0)
    h2 = jnp.dot(a, w2_ref[...], preferred_element_type=jnp.float32) + b2_ref[...]
    s = jnp.sum(h2, axis=0, keepdims=True)
    ss = jnp.sum(h2 * h2, axis=0, keepdims=True)
    upd = jnp.concatenate([s, ss], axis=0)
    maxh2_ref[...] = jnp.max(h2.reshape(BM // N_NEAR, N_NEAR, C_OUT), axis=1)

    @pl.when(i == 0)
    def _():
        stats2_ref[...] = upd

    @pl.when(i > 0)
    def _():
        stats2_ref[...] += upd


def _mlp_pass2(h1, stats1, W2, b2, g1, be1):
    grid = (M_TOTAL // BM,)
    return pl.pallas_call(
        _mlp2_body,
        grid=grid,
        in_specs=[
            pl.BlockSpec((BM, C_MID), lambda i: (i, 0)),
            pl.BlockSpec((2, C_MID), lambda i: (0, 0)),
            pl.BlockSpec((C_MID, C_OUT), lambda i: (0, 0)),
            pl.BlockSpec((1, C_OUT), lambda i: (0, 0)),
            pl.BlockSpec((1, C_MID), lambda i: (0, 0)),
            pl.BlockSpec((1, C_MID), lambda i: (0, 0)),
        ],
        out_specs=[
            pl.BlockSpec((BM // N_NEAR, C_OUT), lambda i: (i, 0)),
            pl.BlockSpec((2, C_OUT), lambda i: (0, 0)),
        ],
        out_shape=[
            jax.ShapeDtypeStruct((B * N_CENTER, C_OUT), jnp.float32),
            jax.ShapeDtypeStruct((2, C_OUT), jnp.float32),
        ],
    )(h1, stats1, W2, b2, g1, be1)


def _epi_body(x_ref, stats2_ref, g2_ref, be2_ref, o_ref):
    m2 = stats2_ref[0:1, :] / N_ROWS_F
    var2 = stats2_ref[1:2, :] / N_ROWS_F - m2 * m2
    inv2 = g2_ref[...] * jax.lax.rsqrt(var2 + EPS)
    o_ref[...] = jnp.maximum((x_ref[...] - m2) * inv2 + be2_ref[...], 0.0)


def _mlp_epilogue(maxh2, stats2, g2, be2):
    grid = (8,)
    bm = (B * N_CENTER) // 8
    return pl.pallas_call(
        _epi_body,
        grid=grid,
        in_specs=[
            pl.BlockSpec((bm, C_OUT), lambda i: (i, 0)),
            pl.BlockSpec((2, C_OUT), lambda i: (0, 0)),
            pl.BlockSpec((1, C_OUT), lambda i: (0, 0)),
            pl.BlockSpec((1, C_OUT), lambda i: (0, 0)),
        ],
        out_specs=pl.BlockSpec((bm, C_OUT), lambda i: (i, 0)),
        out_shape=jax.ShapeDtypeStruct((B * N_CENTER, C_OUT), jnp.float32),
    )(maxh2, stats2, g2, be2)


def kernel(xyz, fea, W1, b1, g1, be1, W2, b2, g2, be2):
    fps_idx, center_xyz = _fps_pallas(xyz)
    sq = jnp.sum(xyz * xyz, axis=-1)
    sqc = jnp.take_along_axis(sq, fps_idx, axis=1)
    BS = B * N_CENTER
    ctrK = jnp.concatenate([
        center_xyz.reshape(BS, 3),
        jnp.ones((BS, 1), jnp.float32),
        sqc.reshape(BS, 1),
        jnp.zeros((BS, 11), jnp.float32),
    ], axis=1)
    xtK = jnp.concatenate([
        -2.0 * jnp.transpose(xyz, (0, 2, 1)),
        sq[:, None, :],
        jnp.ones((B, 1, N_POINT), jnp.float32),
        jnp.zeros((B, 11, N_POINT), jnp.float32),
    ], axis=1)
    negd = _dist_pallas(ctrK, xtK)
    _, group_idx = jax.lax.top_k(negd, N_NEAR)

    gidx = (group_idx.astype(jnp.int32)
            + (jnp.arange(B, dtype=jnp.int32) * N_POINT)[:, None, None]
            ).reshape(M_TOTAL)
    fea2 = fea.reshape(B * N_POINT, C_IN)
    xyzp = jnp.pad(xyz.reshape(B * N_POINT, 3), ((0, 0), (0, 125)))
    table = jnp.concatenate([fea2, xyzp], axis=1)
    gcomb = _sc_gather(gidx, table)

    ctr16 = jnp.pad(center_xyz.reshape(B * N_CENTER, 3), ((0, 0), (0, 13)))
    W1a = W1[:C_IN]
    W1b = jnp.pad(W1[C_IN:], ((0, 13), (0, 0)))
    h1, stats1 = _mlp_pass1(gcomb, ctr16, W1a, W1b, b1[None, :])
    maxh2, stats2 = _mlp_pass2(h1, stats1, W2, b2[None, :], g1[None, :], be1[None, :])
    out = _mlp_epilogue(maxh2, stats2, g2[None, :], be2[None, :])
    return (center_xyz, out.reshape(B, N_CENTER, C_OUT))


# speedup vs baseline: 2.2937x; 1.0004x over previous
import functools

import jax
import jax.numpy as jnp
import numpy as np
from jax import lax
from jax.experimental import pallas as pl
from jax.experimental.pallas import tpu as pltpu
from jax.experimental.pallas import tpu_sc as plsc

B = 4
N_POINT = 4096
N_CENTER = 1024
N_NEAR = 32
C_IN = 128
C_MID = 183
C_OUT = 256

BM = 512
M_TOTAL = B * N_CENTER * N_NEAR
N_ROWS_F = float(M_TOTAL)
EPS = 1e-5


_FPS_SUB = 32
_FPS_LANE = 128


def _fps_body(x_ref, y_ref, z_ref, idx_out_ref, ctr_out_ref):
    X = x_ref[...]
    Y = y_ref[...]
    Z = z_ref[...]
    niota = (jax.lax.broadcasted_iota(jnp.int32, (B, _FPS_SUB, _FPS_LANE), 1) * _FPS_LANE
             + jax.lax.broadcasted_iota(jnp.int32, (B, _FPS_SUB, _FPS_LANE), 2))

    def step(t, carry):
        dist, far = carry
        cmask = niota == far
        cx = jnp.min(jnp.where(cmask, X, 1e9), axis=(1, 2), keepdims=True)
        cy = jnp.min(jnp.where(cmask, Y, 1e9), axis=(1, 2), keepdims=True)
        cz = jnp.min(jnp.where(cmask, Z, 1e9), axis=(1, 2), keepdims=True)
        idx_out_ref[pl.ds(t, 1), :] = far[:, 0, :].reshape(1, B)
        ctr_out_ref[pl.ds(t, 1), :] = jnp.concatenate(
            [cx[:, 0, :], cy[:, 0, :], cz[:, 0, :]], axis=0).reshape(1, 3 * B)
        dx = X - cx
        dy = Y - cy
        dz = Z - cz
        d = (dx * dx + dz * dz) + dy * dy
        dist = jnp.minimum(dist, d)
        m = jnp.max(dist, axis=(1, 2), keepdims=True)
        far = jnp.min(jnp.where(dist == m, niota, jnp.int32(N_POINT)),
                      axis=(1, 2), keepdims=True)
        return dist, far

    init = (jnp.full((B, _FPS_SUB, _FPS_LANE), 1e10, dtype=jnp.float32),
            jnp.zeros((B, 1, 1), dtype=jnp.int32))
    jax.lax.fori_loop(0, N_CENTER, step, init, unroll=False)


def _fps_pallas(xyz):
    xt = jnp.transpose(xyz, (2, 0, 1)).reshape(3, B, _FPS_SUB, _FPS_LANE)
    idx_bs, ctr = pl.pallas_call(
        _fps_body,
        grid=(1,),
        in_specs=[pl.BlockSpec((B, _FPS_SUB, _FPS_LANE), lambda i: (0, 0, 0))] * 3,
        out_specs=[
            pl.BlockSpec((N_CENTER, B), lambda i: (0, 0)),
            pl.BlockSpec((N_CENTER, 3 * B), lambda i: (0, 0)),
        ],
        out_shape=[
            jax.ShapeDtypeStruct((N_CENTER, B), jnp.int32),
            jax.ShapeDtypeStruct((N_CENTER, 3 * B), jnp.float32),
        ],
    )(xt[0], xt[1], xt[2])
    fps_idx = jnp.transpose(idx_bs)
    center_xyz = jnp.transpose(ctr.reshape(N_CENTER, 3, B), (2, 0, 1))
    return fps_idx, center_xyz


def _dist_body(ctr_ref, xt_ref, d_ref):
    lmask = (jax.lax.broadcasted_iota(jnp.int32, (_DCH, 16), 1) < 3
             ).astype(jnp.float32)
    cbf = (ctr_ref[...] * lmask).astype(jnp.bfloat16)
    xbf = xt_ref[0].astype(jnp.bfloat16)
    P2 = jnp.dot(cbf, xbf, preferred_element_type=jnp.float32)
    sqc = ctr_ref[:, 4:5]
    sqrow = xt_ref[0, 3:4, :]
    d_ref[0] = -((sqc + sqrow) + P2)


_DCH = 256


def _dist_pallas(ctrK, xtK):
    return pl.pallas_call(
        _dist_body,
        grid=(B * (N_CENTER // _DCH),),
        in_specs=[
            pl.BlockSpec((_DCH, 16), lambda i: (i, 0)),
            pl.BlockSpec((1, 16, N_POINT), lambda i: (i // 4, 0, 0)),
        ],
        out_specs=pl.BlockSpec((1, _DCH, N_POINT), lambda i: (i // 4, i % 4, 0)),
        out_shape=jax.ShapeDtypeStruct((B, N_CENTER, N_POINT), jnp.float32),
    )(ctrK, xtK)


_GW = 32
_G_PER_W = M_TOTAL // _GW
_GCHUNK = 256
_GN_CHUNKS = _G_PER_W // _GCHUNK
_GD = C_IN + 128


def _sc_gather(gidx, table):
    mesh = plsc.VectorSubcoreMesh(core_axis_name="c", subcore_axis_name="s")

    @functools.partial(
        pl.kernel,
        mesh=mesh,
        out_type=jax.ShapeDtypeStruct((M_TOTAL, _GD), jnp.float32),
        scratch_types=[
            pltpu.VMEM((_GCHUNK,), jnp.int32),
            pltpu.VMEM((_GCHUNK, _GD), jnp.float32),
            pltpu.SemaphoreType.DMA,
        ],
    )
    def body(idx_hbm, tab_hbm, out_hbm, idx_v, rows_v, s1):
        wid = lax.axis_index("s") * 2 + lax.axis_index("c")

        def chunk(c, carry):
            base = wid * _G_PER_W + c * _GCHUNK
            pltpu.sync_copy(idx_hbm.at[pl.ds(base, _GCHUNK)], idx_v)
            pltpu.async_copy(tab_hbm.at[idx_v], rows_v, s1).wait()
            pltpu.sync_copy(rows_v, out_hbm.at[pl.ds(base, _GCHUNK)])
            return carry

        lax.fori_loop(0, _GN_CHUNKS, chunk, 0)

    return body(gidx, table)


def _mlp1_body(g_ref, ctr_ref, w1a_ref, w1b_ref, b1_ref,
               h1_ref, stats_ref):
    i = pl.program_id(0)
    gfea = g_ref[:, :C_IN]
    gx = (g_ref[:, C_IN:C_IN + 16].reshape(BM // N_NEAR, N_NEAR, 16)
          - ctr_ref[...][:, None, :]).reshape(BM, 16)
    h = (jnp.dot(gfea, w1a_ref[...], preferred_element_type=jnp.float32)
         + jnp.dot(gx, w1b_ref[...], preferred_element_type=jnp.float32)
         + b1_ref[...])
    h1_ref[...] = h
    s = jnp.sum(h, axis=0, keepdims=True)
    ss = jnp.sum(h * h, axis=0, keepdims=True)
    upd = jnp.concatenate([s, ss], axis=0)

    @pl.when(i == 0)
    def _():
        stats_ref[...] = upd

    @pl.when(i > 0)
    def _():
        stats_ref[...] += upd


def _mlp_pass1(gcomb, ctr16, W1a, W1b, b1):
    grid = (M_TOTAL // BM,)
    return pl.pallas_call(
        _mlp1_body,
        grid=grid,
        in_specs=[
            pl.BlockSpec((BM, _GD), lambda i: (i, 0)),
            pl.BlockSpec((BM // N_NEAR, 16), lambda i: (i, 0)),
            pl.BlockSpec((C_IN, C_MID), lambda i: (0, 0)),
            pl.BlockSpec((16, C_MID), lambda i: (0, 0)),
            pl.BlockSpec((1, C_MID), lambda i: (0, 0)),
        ],
        out_specs=[
            pl.BlockSpec((BM, C_MID), lambda i: (i, 0)),
            pl.BlockSpec((2, C_MID), lambda i: (0, 0)),
        ],
        out_shape=[
            jax.ShapeDtypeStruct((M_TOTAL, C_MID), jnp.float32),
            jax.ShapeDtypeStruct((2, C_MID), jnp.float32),
        ],
    )(gcomb, ctr16, W1a, W1b, b1)


def _mlp2_body(h1_ref, stats1_ref, w2_ref, b2_ref, g1_ref, be1_ref,
               maxh2_ref, stats2_ref):
    i = pl.program_id(0)
    m1 = stats1_ref[0:1, :] / N_ROWS_F
    var1 = stats1_ref[1:2, :] / N_ROWS_F - m1 * m1
    inv1 = g1_ref[...] * jax.lax.rsqrt(var1 + EPS)
    a = jnp.maximum((h1_ref[...] - m1) * inv1 + be1_ref[...], 0.0)
    h2 = jnp.dot(a, w2_ref[...], preferred_element_type=jnp.float32) + b2_ref[...]
    s = jnp.sum(h2, axis=0, keepdims=True)
    ss = jnp.sum(h2 * h2, axis=0, keepdims=True)
    upd = jnp.concatenate([s, ss], axis=0)
    maxh2_ref[...] = jnp.max(h2.reshape(BM // N_NEAR, N_NEAR, C_OUT), axis=1)

    @pl.when(i == 0)
    def _():
        stats2_ref[...] = upd

    @pl.when(i > 0)
    def _():
        stats2_ref[...] += upd


def _mlp_pass2(h1, stats1, W2, b2, g1, be1):
    grid = (M_TOTAL // BM,)
    return pl.pallas_call(
        _mlp2_body,
        grid=grid,
        in_specs=[
            pl.BlockSpec((BM, C_MID), lambda i: (i, 0)),
            pl.BlockSpec((2, C_MID), lambda i: (0, 0)),
            pl.BlockSpec((C_MID, C_OUT), lambda i: (0, 0)),
            pl.BlockSpec((1, C_OUT), lambda i: (0, 0)),
            pl.BlockSpec((1, C_MID), lambda i: (0, 0)),
            pl.BlockSpec((1, C_MID), lambda i: (0, 0)),
        ],
        out_specs=[
            pl.BlockSpec((BM // N_NEAR, C_OUT), lambda i: (i, 0)),
            pl.BlockSpec((2, C_OUT), lambda i: (0, 0)),
        ],
        out_shape=[
            jax.ShapeDtypeStruct((B * N_CENTER, C_OUT), jnp.float32),
            jax.ShapeDtypeStruct((2, C_OUT), jnp.float32),
        ],
    )(h1, stats1, W2, b2, g1, be1)


def _epi_body(x_ref, stats2_ref, g2_ref, be2_ref, o_ref):
    m2 = stats2_ref[0:1, :] / N_ROWS_F
    var2 = stats2_ref[1:2, :] / N_ROWS_F - m2 * m2
    inv2 = g2_ref[...] * jax.lax.rsqrt(var2 + EPS)
    o_ref[...] = jnp.maximum((x_ref[...] - m2) * inv2 + be2_ref[...], 0.0)


def _mlp_epilogue(maxh2, stats2, g2, be2):
    grid = (8,)
    bm = (B * N_CENTER) // 8
    return pl.pallas_call(
        _epi_body,
        grid=grid,
        in_specs=[
            pl.BlockSpec((bm, C_OUT), lambda i: (i, 0)),
            pl.BlockSpec((2, C_OUT), lambda i: (0, 0)),
            pl.BlockSpec((1, C_OUT), lambda i: (0, 0)),
            pl.BlockSpec((1, C_OUT), lambda i: (0, 0)),
        ],
        out_specs=pl.BlockSpec((bm, C_OUT), lambda i: (i, 0)),
        out_shape=jax.ShapeDtypeStruct((B * N_CENTER, C_OUT), jnp.float32),
    )(maxh2, stats2, g2, be2)


def kernel(xyz, fea, W1, b1, g1, be1, W2, b2, g2, be2):
    fps_idx, center_xyz = _fps_pallas(xyz)
    sq = jnp.sum(xyz * xyz, axis=-1)
    sqc = jnp.take_along_axis(sq, fps_idx, axis=1)
    BS = B * N_CENTER
    ctrK = jnp.concatenate([
        center_xyz.reshape(BS, 3),
        jnp.ones((BS, 1), jnp.float32),
        sqc.reshape(BS, 1),
        jnp.zeros((BS, 11), jnp.float32),
    ], axis=1)
    xtK = jnp.concatenate([
        -2.0 * jnp.transpose(xyz, (0, 2, 1)),
        sq[:, None, :],
        jnp.ones((B, 1, N_POINT), jnp.float32),
        jnp.zeros((B, 11, N_POINT), jnp.float32),
    ], axis=1)
    negd = _dist_pallas(ctrK, xtK)
    _, group_idx = jax.lax.top_k(negd, N_NEAR)

    gidx = (group_idx.astype(jnp.int32)
            + (jnp.arange(B, dtype=jnp.int32) * N_POINT)[:, None, None]
            ).reshape(M_TOTAL)
    fea2 = fea.reshape(B * N_POINT, C_IN)
    xyzp = jnp.pad(xyz.reshape(B * N_POINT, 3), ((0, 0), (0, 125)))
    table = jnp.concatenate([fea2, xyzp], axis=1)
    gcomb = _sc_gather(gidx, table)

    ctr16 = jnp.pad(center_xyz.reshape(B * N_CENTER, 3), ((0, 0), (0, 13)))
    W1a = W1[:C_IN]
    W1b = jnp.pad(W1[C_IN:], ((0, 13), (0, 0)))
    h1, stats1 = _mlp_pass1(gcomb, ctr16, W1a, W1b, b1[None, :])
    maxh2, stats2 = _mlp_pass2(h1, stats1, W2, b2[None, :], g1[None, :], be1[None, :])
    out = _mlp_epilogue(maxh2, stats2, g2[None, :], be2[None, :])
    return (center_xyz, out.reshape(B, N_CENTER, C_OUT))


# speedup vs baseline: 2.2940x; 1.0001x over previous
import functools

import jax
import jax.numpy as jnp
from jax import lax
from jax.experimental import pallas as pl
from jax.experimental.pallas import tpu as pltpu
from jax.experimental.pallas import tpu_sc as plsc

B = 4
N_POINT = 4096
N_CENTER = 1024
N_NEAR = 32
C_IN = 128
C_MID = 183
C_OUT = 256

BM = 512
M_TOTAL = B * N_CENTER * N_NEAR
N_ROWS_F = float(M_TOTAL)
EPS = 1e-5


_FPS_SUB = 32
_FPS_LANE = 128


def _fps_body(x_ref, y_ref, z_ref, idx_out_ref, ctr_out_ref):
    X = x_ref[...]
    Y = y_ref[...]
    Z = z_ref[...]
    niota = (jax.lax.broadcasted_iota(jnp.int32, (B, _FPS_SUB, _FPS_LANE), 1) * _FPS_LANE
             + jax.lax.broadcasted_iota(jnp.int32, (B, _FPS_SUB, _FPS_LANE), 2))

    def step(t, carry):
        dist, far = carry
        cmask = niota == far
        cx = jnp.min(jnp.where(cmask, X, 1e9), axis=(1, 2), keepdims=True)
        cy = jnp.min(jnp.where(cmask, Y, 1e9), axis=(1, 2), keepdims=True)
        cz = jnp.min(jnp.where(cmask, Z, 1e9), axis=(1, 2), keepdims=True)
        idx_out_ref[pl.ds(t, 1), :] = far[:, 0, :].reshape(1, B)
        ctr_out_ref[pl.ds(t, 1), :] = jnp.concatenate(
            [cx[:, 0, :], cy[:, 0, :], cz[:, 0, :]], axis=0).reshape(1, 3 * B)
        dx = X - cx
        dy = Y - cy
        dz = Z - cz
        d = (dx * dx + dz * dz) + dy * dy
        dist = jnp.minimum(dist, d)
        m = jnp.max(dist, axis=(1, 2), keepdims=True)
        far = jnp.min(jnp.where(dist == m, niota, jnp.int32(N_POINT)),
                      axis=(1, 2), keepdims=True)
        return dist, far

    init = (jnp.full((B, _FPS_SUB, _FPS_LANE), 1e10, dtype=jnp.float32),
            jnp.zeros((B, 1, 1), dtype=jnp.int32))
    jax.lax.fori_loop(0, N_CENTER, step, init, unroll=False)


def _fps_pallas(xyz):
    xt = jnp.transpose(xyz, (2, 0, 1)).reshape(3, B, _FPS_SUB, _FPS_LANE)
    idx_bs, ctr = pl.pallas_call(
        _fps_body,
        grid=(1,),
        in_specs=[pl.BlockSpec((B, _FPS_SUB, _FPS_LANE), lambda i: (0, 0, 0))] * 3,
        out_specs=[
            pl.BlockSpec((N_CENTER, B), lambda i: (0, 0)),
            pl.BlockSpec((N_CENTER, 3 * B), lambda i: (0, 0)),
        ],
        out_shape=[
            jax.ShapeDtypeStruct((N_CENTER, B), jnp.int32),
            jax.ShapeDtypeStruct((N_CENTER, 3 * B), jnp.float32),
        ],
    )(xt[0], xt[1], xt[2])
    fps_idx = jnp.transpose(idx_bs)
    center_xyz = jnp.transpose(ctr.reshape(N_CENTER, 3, B), (2, 0, 1))
    return fps_idx, center_xyz


def _dist_body(ctr_ref, xt_ref, d_ref):
    lmask = (jax.lax.broadcasted_iota(jnp.int32, (_DCH, 16), 1) < 3
             ).astype(jnp.float32)
    cbf = (ctr_ref[...] * lmask).astype(jnp.bfloat16)
    xbf = xt_ref[0].astype(jnp.bfloat16)
    P2 = jnp.dot(cbf, xbf, preferred_element_type=jnp.float32)
    sqc = ctr_ref[:, 4:5]
    sqrow = xt_ref[0, 3:4, :]
    d_ref[0] = -((sqc + sqrow) + P2)


_DCH = 256


def _dist_pallas(ctrK, xtK):
    return pl.pallas_call(
        _dist_body,
        grid=(B * (N_CENTER // _DCH),),
        in_specs=[
            pl.BlockSpec((_DCH, 16), lambda i: (i, 0)),
            pl.BlockSpec((1, 16, N_POINT), lambda i: (i // 4, 0, 0)),
        ],
        out_specs=pl.BlockSpec((1, _DCH, N_POINT), lambda i: (i // 4, i % 4, 0)),
        out_shape=jax.ShapeDtypeStruct((B, N_CENTER, N_POINT), jnp.float32),
    )(ctrK, xtK)


_GW = 32
_G_PER_W = M_TOTAL // _GW
_GCHUNK = 256
_GN_CHUNKS = _G_PER_W // _GCHUNK
_GD = C_IN + 128


def _sc_gather(gidx, table):
    mesh = plsc.VectorSubcoreMesh(core_axis_name="c", subcore_axis_name="s")

    @functools.partial(
        pl.kernel,
        mesh=mesh,
        out_type=jax.ShapeDtypeStruct((M_TOTAL, _GD), jnp.float32),
        scratch_types=[
            pltpu.VMEM((_GCHUNK,), jnp.int32),
            pltpu.VMEM((_GCHUNK, _GD), jnp.float32),
            pltpu.SemaphoreType.DMA,
        ],
    )
    def body(idx_hbm, tab_hbm, out_hbm, idx_v, rows_v, s1):
        wid = lax.axis_index("s") * 2 + lax.axis_index("c")

        def chunk(c, carry):
            base = wid * _G_PER_W + c * _GCHUNK
            pltpu.sync_copy(idx_hbm.at[pl.ds(base, _GCHUNK)], idx_v)
            pltpu.async_copy(tab_hbm.at[idx_v], rows_v, s1).wait()
            pltpu.sync_copy(rows_v, out_hbm.at[pl.ds(base, _GCHUNK)])
            return carry

        lax.fori_loop(0, _GN_CHUNKS, chunk, 0)

    return body(gidx, table)


def _mlp1_body(g_ref, ctr_ref, w1a_ref, w1b_ref, b1_ref,
               h1_ref, stats_ref):
    i = pl.program_id(0)
    gfea = g_ref[:, :C_IN]
    gx = (g_ref[:, C_IN:C_IN + 16].reshape(BM // N_NEAR, N_NEAR, 16)
          - ctr_ref[...][:, None, :]).reshape(BM, 16)
    h = (jnp.dot(gfea, w1a_ref[...], preferred_element_type=jnp.float32)
         + jnp.dot(gx, w1b_ref[...], preferred_element_type=jnp.float32)
         + b1_ref[...])
    h1_ref[...] = h
    s = jnp.sum(h, axis=0, keepdims=True)
    ss = jnp.sum(h * h, axis=0, keepdims=True)
    upd = jnp.concatenate([s, ss], axis=0)

    @pl.when(i == 0)
    def _():
        stats_ref[...] = upd

    @pl.when(i > 0)
    def _():
        stats_ref[...] += upd


def _mlp_pass1(gcomb, ctr16, W1a, W1b, b1):
    grid = (M_TOTAL // BM,)
    return pl.pallas_call(
        _mlp1_body,
        grid=grid,
        in_specs=[
            pl.BlockSpec((BM, _GD), lambda i: (i, 0)),
            pl.BlockSpec((BM // N_NEAR, 16), lambda i: (i, 0)),
            pl.BlockSpec((C_IN, C_MID), lambda i: (0, 0)),
            pl.BlockSpec((16, C_MID), lambda i: (0, 0)),
            pl.BlockSpec((1, C_MID), lambda i: (0, 0)),
        ],
        out_specs=[
            pl.BlockSpec((BM, C_MID), lambda i: (i, 0)),
            pl.BlockSpec((2, C_MID), lambda i: (0, 0)),
        ],
        out_shape=[
            jax.ShapeDtypeStruct((M_TOTAL, C_MID), jnp.float32),
            jax.ShapeDtypeStruct((2, C_MID), jnp.float32),
        ],
    )(gcomb, ctr16, W1a, W1b, b1)


def _mlp2_body(h1_ref, stats1_ref, w2_ref, b2_ref, g1_ref, be1_ref,
               maxh2_ref, stats2_ref):
    i = pl.program_id(0)
    m1 = stats1_ref[0:1, :] / N_ROWS_F
    var1 = stats1_ref[1:2, :] / N_ROWS_F - m1 * m1
    inv1 = g1_ref[...] * jax.lax.rsqrt(var1 + EPS)
    a = jnp.maximum((h1_ref[...] - m1) * inv1 + be1_ref[...], 0.0)
    h2 = jnp.dot(a, w2_ref[...], preferred_element_type=jnp.float32) + b2_ref[...]
    s = jnp.sum(h2, axis=0, keepdims=True)
    ss = jnp.sum(h2 * h2, axis=0, keepdims=True)
    upd = jnp.concatenate([s, ss], axis=0)
    maxh2_ref[...] = jnp.max(h2.reshape(BM // N_NEAR, N_NEAR, C_OUT), axis=1)

    @pl.when(i == 0)
    def _():
        stats2_ref[...] = upd

    @pl.when(i > 0)
    def _():
        stats2_ref[...] += upd


def _mlp_pass2(h1, stats1, W2, b2, g1, be1):
    grid = (M_TOTAL // BM,)
    return pl.pallas_call(
        _mlp2_body,
        grid=grid,
        in_specs=[
            pl.BlockSpec((BM, C_MID), lambda i: (i, 0)),
            pl.BlockSpec((2, C_MID), lambda i: (0, 0)),
            pl.BlockSpec((C_MID, C_OUT), lambda i: (0, 0)),
            pl.BlockSpec((1, C_OUT), lambda i: (0, 0)),
            pl.BlockSpec((1, C_MID), lambda i: (0, 0)),
            pl.BlockSpec((1, C_MID), lambda i: (0, 0)),
        ],
        out_specs=[
            pl.BlockSpec((BM // N_NEAR, C_OUT), lambda i: (i, 0)),
            pl.BlockSpec((2, C_OUT), lambda i: (0, 0)),
        ],
        out_shape=[
            jax.ShapeDtypeStruct((B * N_CENTER, C_OUT), jnp.float32),
            jax.ShapeDtypeStruct((2, C_OUT), jnp.float32),
        ],
    )(h1, stats1, W2, b2, g1, be1)


def _epi_body(x_ref, stats2_ref, g2_ref, be2_ref, o_ref):
    m2 = stats2_ref[0:1, :] / N_ROWS_F
    var2 = stats2_ref[1:2, :] / N_ROWS_F - m2 * m2
    inv2 = g2_ref[...] * jax.lax.rsqrt(var2 + EPS)
    o_ref[...] = jnp.maximum((x_ref[...] - m2) * inv2 + be2_ref[...], 0.0)


def _mlp_epilogue(maxh2, stats2, g2, be2):
    grid = (8,)
    bm = (B * N_CENTER) // 8
    return pl.pallas_call(
        _epi_body,
        grid=grid,
        in_specs=[
            pl.BlockSpec((bm, C_OUT), lambda i: (i, 0)),
            pl.BlockSpec((2, C_OUT), lambda i: (0, 0)),
            pl.BlockSpec((1, C_OUT), lambda i: (0, 0)),
            pl.BlockSpec((1, C_OUT), lambda i: (0, 0)),
        ],
        out_specs=pl.BlockSpec((bm, C_OUT), lambda i: (i, 0)),
        out_shape=jax.ShapeDtypeStruct((B * N_CENTER, C_OUT), jnp.float32),
    )(maxh2, stats2, g2, be2)


def kernel(xyz, fea, W1, b1, g1, be1, W2, b2, g2, be2):
    fps_idx, center_xyz = _fps_pallas(xyz)
    sq = jnp.sum(xyz * xyz, axis=-1)
    sqc = jnp.take_along_axis(sq, fps_idx, axis=1)
    BS = B * N_CENTER
    ctrK = jnp.concatenate([
        center_xyz.reshape(BS, 3),
        jnp.ones((BS, 1), jnp.float32),
        sqc.reshape(BS, 1),
        jnp.zeros((BS, 11), jnp.float32),
    ], axis=1)
    xtK = jnp.concatenate([
        -2.0 * jnp.transpose(xyz, (0, 2, 1)),
        sq[:, None, :],
        jnp.ones((B, 1, N_POINT), jnp.float32),
        jnp.zeros((B, 11, N_POINT), jnp.float32),
    ], axis=1)
    negd = _dist_pallas(ctrK, xtK)
    _, group_idx = jax.lax.top_k(negd, N_NEAR)

    gidx = (group_idx.astype(jnp.int32)
            + (jnp.arange(B, dtype=jnp.int32) * N_POINT)[:, None, None]
            ).reshape(M_TOTAL)
    fea2 = fea.reshape(B * N_POINT, C_IN)
    xyzp = jnp.pad(xyz.reshape(B * N_POINT, 3), ((0, 0), (0, 125)))
    table = jnp.concatenate([fea2, xyzp], axis=1)
    gcomb = _sc_gather(gidx, table)

    ctr16 = jnp.pad(center_xyz.reshape(B * N_CENTER, 3), ((0, 0), (0, 13)))
    W1a = W1[:C_IN]
    W1b = jnp.pad(W1[C_IN:], ((0, 13), (0, 0)))
    h1, stats1 = _mlp_pass1(gcomb, ctr16, W1a, W1b, b1[None, :])
    maxh2, stats2 = _mlp_pass2(h1, stats1, W2, b2[None, :], g1[None, :], be1[None, :])
    out = _mlp_epilogue(maxh2, stats2, g2[None, :], be2[None, :])
    return (center_xyz, out.reshape(B, N_CENTER, C_OUT))
